# Initial kernel scaffold; baseline (speedup 1.0000x reference)
import jax
import jax.numpy as jnp
from jax.experimental import pallas as pl


def kernel(x, p_embeddings):
    raise NotImplementedError("write your pallas kernel here")


# speedup vs baseline: 3.2333x; 3.2333x over previous
import jax
import jax.numpy as jnp
from jax.experimental import pallas as pl
from jax.experimental.pallas import tpu as pltpu

_BS = 256


def _add_body(x_ref, p_ref, o_ref):
    o_ref[...] = x_ref[...] + p_ref[...][None, :, :]


def kernel(x, p_embeddings):
    b, s, d = x.shape
    n_blocks = s // _BS
    return pl.pallas_call(
        _add_body,
        grid=(n_blocks,),
        in_specs=[
            pl.BlockSpec((b, _BS, d), lambda i: (0, i, 0)),
            pl.BlockSpec((_BS, d), lambda i: (i, 0)),
        ],
        out_specs=pl.BlockSpec((b, _BS, d), lambda i: (0, i, 0)),
        out_shape=jax.ShapeDtypeStruct((b, s, d), x.dtype),
    )(x, p_embeddings)
